# baseline (speedup 1.0000x reference)
import jax
import jax.numpy as jnp
from jax.experimental import pallas as pl
import jax.experimental.pallas.tpu as pltpu
from jax.experimental.pallas import tpu_sc as plsc

T = 2048
D = 1024
E = 8
FF = 1024
TT = 256
NT = T // TT
TTC = 128
NTILES = 40
NSLOT = NTILES * TTC
NA = 2 * T

_SC_INFO = plsc.get_sparse_core_info()
_NC, _NS = _SC_INFO.num_cores, _SC_INFO.num_subcores
_NW = _NC * _NS


def _router_body(x_ref, scale_ref, gk_ref, gb_ref,
                 normed_ref, mf_ref, pos_ref, te_ref, used_ref,
                 mi_s, base_s, l_s):
    t = pl.program_id(0)

    @pl.when(t == 0)
    def _init():
        base_s[...] = jnp.zeros_like(base_s)
        row = jax.lax.broadcasted_iota(jnp.int32, (TT, TT), 0)
        col = jax.lax.broadcasted_iota(jnp.int32, (TT, TT), 1)
        l_s[...] = (row > col).astype(jnp.float32)

    xt = x_ref[...]
    rms = jnp.sqrt(jnp.mean(xt * xt, axis=-1, keepdims=True) + 1e-5)
    normed = (xt / rms) * scale_ref[0, :]
    normed_ref[...] = normed

    logits = jax.lax.dot_general(
        normed, gk_ref[...], (((1,), (0,)), ((), ())),
        preferred_element_type=jnp.float32) + gb_ref[0, :]
    iota = jax.lax.broadcasted_iota(jnp.int32, (TT, E), 1)
    m1 = jnp.max(logits, axis=-1, keepdims=True)
    idx1 = jnp.min(jnp.where(logits == m1, iota, E), axis=-1, keepdims=True)
    masked = jnp.where(iota == idx1, -jnp.inf, logits)
    m2 = jnp.max(masked, axis=-1, keepdims=True)
    idx2 = jnp.min(jnp.where(masked == m2, iota, E), axis=-1, keepdims=True)
    e2 = jnp.exp(m2 - m1)
    denom = 1.0 + e2
    wa = 1.0 / denom
    wb = e2 / denom

    a0 = (iota == idx1).astype(jnp.float32)
    a1 = (iota == idx2).astype(jnp.float32)
    ctile = a0 + a1
    prefix = jax.lax.dot_general(
        l_s[...], ctile, (((1,), (0,)), ((), ())),
        preferred_element_type=jnp.float32) + base_s[...]
    r0 = jnp.sum(a0 * prefix, axis=-1, keepdims=True)
    r1 = jnp.sum(a1 * prefix, axis=-1, keepdims=True)

    mi_s[pl.ds(t * TT, TT), :] = jnp.concatenate(
        [idx1, idx2, r0.astype(jnp.int32), r1.astype(jnp.int32)], axis=1)
    mf_ref[...] = jnp.concatenate([wa, wb], axis=1)
    base_s[...] = base_s[...] + jnp.sum(ctile, axis=0, keepdims=True)

    @pl.when(t == NT - 1)
    def _dispatch():
        cnt = base_s[...]
        ntile = jnp.floor((cnt + (TTC - 1)) * (1.0 / TTC))
        erow = jax.lax.broadcasted_iota(jnp.int32, (E, E), 0)
        ecol = jax.lax.broadcasted_iota(jnp.int32, (E, E), 1)
        ustrict = (erow < ecol).astype(jnp.float32)
        tb = jax.lax.dot_general(
            ntile, ustrict, (((1,), (0,)), ((), ())),
            preferred_element_type=jnp.float32)
        off = tb * TTC

        ai1 = mi_s[:, 0:1]
        ai2 = mi_s[:, 1:2]
        ar0 = mi_s[:, 2:3]
        ar1 = mi_s[:, 3:4]
        tiota = jax.lax.broadcasted_iota(jnp.int32, (T, E), 1)
        b0 = (tiota == ai1).astype(jnp.float32)
        b1 = (tiota == ai2).astype(jnp.float32)
        pos0 = jnp.sum(b0 * off, axis=-1, keepdims=True).astype(jnp.int32) + ar0
        pos1 = jnp.sum(b1 * off, axis=-1, keepdims=True).astype(jnp.int32) + ar1
        pos_ref[...] = jnp.concatenate([pos0, pos1], axis=1)

        jt = jax.lax.broadcasted_iota(jnp.int32, (NTILES, E), 0).astype(jnp.float32)
        inb = jnp.where((jt >= tb) & (jt < tb + ntile), 1.0, 0.0)
        eid = jax.lax.broadcasted_iota(jnp.int32, (NTILES, E), 1).astype(jnp.float32)
        te = jnp.sum(inb * eid, axis=-1, keepdims=True) + \
            (1.0 - jnp.sum(inb, axis=-1, keepdims=True)) * (E - 1)
        te_ref[...] = te.astype(jnp.int32)
        used_ref[...] = jnp.sum(
            ntile, axis=-1, keepdims=True).astype(jnp.int32)


def _router(x, scale, gate_kernel, gate_bias):
    return pl.pallas_call(
        _router_body,
        grid=(NT,),
        in_specs=[
            pl.BlockSpec((TT, D), lambda t: (t, 0)),
            pl.BlockSpec((1, D), lambda t: (0, 0)),
            pl.BlockSpec((D, E), lambda t: (0, 0)),
            pl.BlockSpec((1, E), lambda t: (0, 0)),
        ],
        out_specs=[
            pl.BlockSpec((TT, D), lambda t: (t, 0)),
            pl.BlockSpec((TT, 2), lambda t: (t, 0)),
            pl.BlockSpec((T, 2), lambda t: (0, 0)),
            pl.BlockSpec((NTILES, 1), lambda t: (0, 0)),
            pl.BlockSpec((1, 1), lambda t: (0, 0)),
        ],
        out_shape=[
            jax.ShapeDtypeStruct((T, D), jnp.float32),
            jax.ShapeDtypeStruct((T, 2), jnp.float32),
            jax.ShapeDtypeStruct((T, 2), jnp.int32),
            jax.ShapeDtypeStruct((NTILES, 1), jnp.int32),
            jax.ShapeDtypeStruct((1, 1), jnp.int32),
        ],
        scratch_shapes=[
            pltpu.VMEM((T, 4), jnp.int32),
            pltpu.VMEM((1, E), jnp.float32),
            pltpu.VMEM((TT, TT), jnp.float32),
        ],
    )(x, scale.reshape(1, D), gate_kernel, gate_bias.reshape(1, E))


def _scatter_rows(normed, p0, p1):
    b_per_w = T // _NW
    mesh = plsc.VectorSubcoreMesh(core_axis_name="c", subcore_axis_name="s")

    def body(normed_hbm, p0_hbm, p1_hbm, gx_hbm, rows_v, idx_v, sem):
        wid = jax.lax.axis_index("s") * _NC + jax.lax.axis_index("c")
        base = wid * b_per_w
        pltpu.sync_copy(normed_hbm.at[pl.ds(base, b_per_w)], rows_v)
        pltpu.sync_copy(p0_hbm.at[pl.ds(base, b_per_w)], idx_v)
        pltpu.async_copy(rows_v, gx_hbm.at[idx_v], sem).wait()
        pltpu.sync_copy(p1_hbm.at[pl.ds(base, b_per_w)], idx_v)
        pltpu.async_copy(rows_v, gx_hbm.at[idx_v], sem).wait()

    f = pl.kernel(
        body,
        out_type=jax.ShapeDtypeStruct((NSLOT, D), jnp.float32),
        mesh=mesh,
        scratch_types=[
            pltpu.VMEM((b_per_w, D), jnp.float32),
            pltpu.VMEM((b_per_w,), jnp.int32),
            pltpu.SemaphoreType.DMA,
        ],
    )
    return f(normed, p0, p1)


def _gather_rows(table, idx, nrows_total, sub=64):
    b_per_w = nrows_total // _NW
    nsub = b_per_w // sub
    mesh = plsc.VectorSubcoreMesh(core_axis_name="c", subcore_axis_name="s")

    def body(table_hbm, idx_hbm, out_hbm, idx_v, rows_v, sem):
        wid = jax.lax.axis_index("s") * _NC + jax.lax.axis_index("c")
        base = wid * b_per_w
        for k in range(nsub):
            pltpu.sync_copy(idx_hbm.at[pl.ds(base + k * sub, sub)], idx_v)
            pltpu.async_copy(table_hbm.at[idx_v], rows_v, sem).wait()
            pltpu.sync_copy(rows_v, out_hbm.at[pl.ds(base + k * sub, sub)])

    f = pl.kernel(
        body,
        out_type=jax.ShapeDtypeStruct((nrows_total, D), jnp.float32),
        mesh=mesh,
        scratch_types=[
            pltpu.VMEM((sub,), jnp.int32),
            pltpu.VMEM((sub, D), jnp.float32),
            pltpu.SemaphoreType.DMA,
        ],
    )
    return f(table, idx)


def _expert_body(te_ref, used_ref, gx_ref, w1_ref, b1_ref, w2_ref, b2_ref,
                 o_ref):
    j = pl.program_id(0)

    @pl.when(j < used_ref[0, 0])
    def _compute():
        gx = gx_ref[...]
        m1out = jax.lax.dot_general(
            gx, w1_ref[0], (((1,), (1,)), ((), ())),
            preferred_element_type=jnp.float32) + b1_ref[0, 0, :]
        gate_part = jnp.minimum(m1out[:, :FF], 7.0)
        linear_part = jnp.clip(m1out[:, FF:], -7.0, 7.0)
        swish_gate = gate_part * jax.nn.sigmoid(1.702 * gate_part)
        activated = swish_gate * (linear_part + 1.0)
        o_ref[...] = jax.lax.dot_general(
            activated, w2_ref[0], (((1,), (1,)), ((), ())),
            preferred_element_type=jnp.float32) + b2_ref[0, 0, :]


def _experts(te, used, gx, mlp1_weight, mlp1_bias, mlp2_weight, mlp2_bias):
    grid_spec = pltpu.PrefetchScalarGridSpec(
        num_scalar_prefetch=2,
        grid=(NTILES,),
        in_specs=[
            pl.BlockSpec((TTC, D), lambda j, te, u: (j, 0)),
            pl.BlockSpec((1, 2 * FF, D), lambda j, te, u: (te[j, 0], 0, 0)),
            pl.BlockSpec((1, 1, 2 * FF), lambda j, te, u: (te[j, 0], 0, 0)),
            pl.BlockSpec((1, D, FF), lambda j, te, u: (te[j, 0], 0, 0)),
            pl.BlockSpec((1, 1, D), lambda j, te, u: (te[j, 0], 0, 0)),
        ],
        out_specs=pl.BlockSpec((TTC, D), lambda j, te, u: (j, 0)),
    )
    return pl.pallas_call(
        _expert_body,
        grid_spec=grid_spec,
        out_shape=jax.ShapeDtypeStruct((NSLOT, D), jnp.float32),
    )(te, used, gx,
      mlp1_weight, mlp1_bias.reshape(E, 1, 2 * FF),
      mlp2_weight, mlp2_bias.reshape(E, 1, D))


def _combine_body(x_ref, mf_ref, ga_ref, gb_ref, o_ref):
    wa = mf_ref[:, 0:1]
    wb = mf_ref[:, 1:2]
    o_ref[...] = x_ref[...] + wa * ga_ref[...] + wb * gb_ref[...]


def _combine(x, mf, gpair):
    return pl.pallas_call(
        _combine_body,
        grid=(NT,),
        in_specs=[
            pl.BlockSpec((TT, D), lambda t: (t, 0)),
            pl.BlockSpec((TT, 2), lambda t: (t, 0)),
            pl.BlockSpec((TT, D), lambda t: (t, 0)),
            pl.BlockSpec((TT, D), lambda t: (NT + t, 0)),
        ],
        out_specs=pl.BlockSpec((TT, D), lambda t: (t, 0)),
        out_shape=jax.ShapeDtypeStruct((T, D), jnp.float32),
    )(x, mf, gpair, gpair)


@jax.jit
def kernel(x, scale, gate_kernel, gate_bias, mlp1_weight, mlp1_bias,
           mlp2_weight, mlp2_bias):
    normed, mf, pos, te, used = _router(x, scale, gate_kernel, gate_bias)
    p0 = pos[:, 0]
    p1 = pos[:, 1]
    gx = _scatter_rows(normed, p0, p1)
    oslots = _experts(te, used, gx,
                      mlp1_weight, mlp1_bias, mlp2_weight, mlp2_bias)
    gpair = _gather_rows(oslots, jnp.concatenate([p0, p1]), NA)
    return _combine(x, mf, gpair)


# speedup vs baseline: 1.2682x; 1.2682x over previous
import jax
import jax.numpy as jnp
from jax.experimental import pallas as pl
import jax.experimental.pallas.tpu as pltpu
from jax.experimental.pallas import tpu_sc as plsc

T = 2048
D = 1024
E = 8
FF = 1024
TT = 256
NT = T // TT
TTC = 256
NTILES = 24
NSLOT = NTILES * TTC
NA = 2 * T

_SC_INFO = plsc.get_sparse_core_info()
_NC, _NS = _SC_INFO.num_cores, _SC_INFO.num_subcores
_NW = _NC * _NS


def _router_body(x_ref, scale_ref, gk_ref, gb_ref,
                 normed_ref, mf_ref, pos_ref, te_ref, used_ref,
                 mi_s, base_s, l_s):
    t = pl.program_id(0)

    @pl.when(t == 0)
    def _init():
        base_s[...] = jnp.zeros_like(base_s)
        row = jax.lax.broadcasted_iota(jnp.int32, (TT, TT), 0)
        col = jax.lax.broadcasted_iota(jnp.int32, (TT, TT), 1)
        l_s[...] = (row > col).astype(jnp.float32)

    xt = x_ref[...]
    rms = jnp.sqrt(jnp.mean(xt * xt, axis=-1, keepdims=True) + 1e-5)
    normed = (xt / rms) * scale_ref[0, :]
    normed_ref[...] = normed

    logits = jax.lax.dot_general(
        normed, gk_ref[...], (((1,), (0,)), ((), ())),
        preferred_element_type=jnp.float32) + gb_ref[0, :]
    iota = jax.lax.broadcasted_iota(jnp.int32, (TT, E), 1)
    m1 = jnp.max(logits, axis=-1, keepdims=True)
    idx1 = jnp.min(jnp.where(logits == m1, iota, E), axis=-1, keepdims=True)
    masked = jnp.where(iota == idx1, -jnp.inf, logits)
    m2 = jnp.max(masked, axis=-1, keepdims=True)
    idx2 = jnp.min(jnp.where(masked == m2, iota, E), axis=-1, keepdims=True)
    e2 = jnp.exp(m2 - m1)
    denom = 1.0 + e2
    wa = 1.0 / denom
    wb = e2 / denom

    a0 = (iota == idx1).astype(jnp.float32)
    a1 = (iota == idx2).astype(jnp.float32)
    ctile = a0 + a1
    prefix = jax.lax.dot_general(
        l_s[...], ctile, (((1,), (0,)), ((), ())),
        preferred_element_type=jnp.float32) + base_s[...]
    r0 = jnp.sum(a0 * prefix, axis=-1, keepdims=True)
    r1 = jnp.sum(a1 * prefix, axis=-1, keepdims=True)

    mi_s[pl.ds(t * TT, TT), :] = jnp.concatenate(
        [idx1, idx2, r0.astype(jnp.int32), r1.astype(jnp.int32)], axis=1)
    mf_ref[...] = jnp.concatenate([wa, wb], axis=1)
    base_s[...] = base_s[...] + jnp.sum(ctile, axis=0, keepdims=True)

    @pl.when(t == NT - 1)
    def _dispatch():
        cnt = base_s[...]
        ntile = jnp.floor((cnt + (TTC - 1)) * (1.0 / TTC))
        erow = jax.lax.broadcasted_iota(jnp.int32, (E, E), 0)
        ecol = jax.lax.broadcasted_iota(jnp.int32, (E, E), 1)
        ustrict = (erow < ecol).astype(jnp.float32)
        tb = jax.lax.dot_general(
            ntile, ustrict, (((1,), (0,)), ((), ())),
            preferred_element_type=jnp.float32)
        off = tb * TTC

        ai1 = mi_s[:, 0:1]
        ai2 = mi_s[:, 1:2]
        ar0 = mi_s[:, 2:3]
        ar1 = mi_s[:, 3:4]
        tiota = jax.lax.broadcasted_iota(jnp.int32, (T, E), 1)
        b0 = (tiota == ai1).astype(jnp.float32)
        b1 = (tiota == ai2).astype(jnp.float32)
        pos0 = jnp.sum(b0 * off, axis=-1, keepdims=True).astype(jnp.int32) + ar0
        pos1 = jnp.sum(b1 * off, axis=-1, keepdims=True).astype(jnp.int32) + ar1
        pos_ref[...] = jnp.concatenate([pos0, pos1], axis=1)

        jt = jax.lax.broadcasted_iota(jnp.int32, (NTILES, E), 0).astype(jnp.float32)
        inb = jnp.where((jt >= tb) & (jt < tb + ntile), 1.0, 0.0)
        eid = jax.lax.broadcasted_iota(jnp.int32, (NTILES, E), 1).astype(jnp.float32)
        te = jnp.sum(inb * eid, axis=-1, keepdims=True) + \
            (1.0 - jnp.sum(inb, axis=-1, keepdims=True)) * (E - 1)
        te_ref[...] = te.astype(jnp.int32)
        used_ref[...] = jnp.sum(
            ntile, axis=-1, keepdims=True).astype(jnp.int32)


def _router(x, scale, gate_kernel, gate_bias):
    return pl.pallas_call(
        _router_body,
        grid=(NT,),
        in_specs=[
            pl.BlockSpec((TT, D), lambda t: (t, 0)),
            pl.BlockSpec((1, D), lambda t: (0, 0)),
            pl.BlockSpec((D, E), lambda t: (0, 0)),
            pl.BlockSpec((1, E), lambda t: (0, 0)),
        ],
        out_specs=[
            pl.BlockSpec((TT, D), lambda t: (t, 0)),
            pl.BlockSpec((TT, 2), lambda t: (t, 0)),
            pl.BlockSpec((T, 2), lambda t: (0, 0)),
            pl.BlockSpec((NTILES, 1), lambda t: (0, 0)),
            pl.BlockSpec((1, 1), lambda t: (0, 0)),
        ],
        out_shape=[
            jax.ShapeDtypeStruct((T, D), jnp.float32),
            jax.ShapeDtypeStruct((T, 2), jnp.float32),
            jax.ShapeDtypeStruct((T, 2), jnp.int32),
            jax.ShapeDtypeStruct((NTILES, 1), jnp.int32),
            jax.ShapeDtypeStruct((1, 1), jnp.int32),
        ],
        scratch_shapes=[
            pltpu.VMEM((T, 4), jnp.int32),
            pltpu.VMEM((1, E), jnp.float32),
            pltpu.VMEM((TT, TT), jnp.float32),
        ],
    )(x, scale.reshape(1, D), gate_kernel, gate_bias.reshape(1, E))


def _scatter_rows(normed, p0, p1):
    b_per_w = T // _NW
    mesh = plsc.VectorSubcoreMesh(core_axis_name="c", subcore_axis_name="s")

    def body(normed_hbm, p0_hbm, p1_hbm, gx_hbm, rows_v, idx_v, sem):
        wid = jax.lax.axis_index("s") * _NC + jax.lax.axis_index("c")
        base = wid * b_per_w
        pltpu.sync_copy(normed_hbm.at[pl.ds(base, b_per_w)], rows_v)
        pltpu.sync_copy(p0_hbm.at[pl.ds(base, b_per_w)], idx_v)
        pltpu.async_copy(rows_v, gx_hbm.at[idx_v], sem).wait()
        pltpu.sync_copy(p1_hbm.at[pl.ds(base, b_per_w)], idx_v)
        pltpu.async_copy(rows_v, gx_hbm.at[idx_v], sem).wait()

    f = pl.kernel(
        body,
        out_type=jax.ShapeDtypeStruct((NSLOT, D), jnp.float32),
        mesh=mesh,
        scratch_types=[
            pltpu.VMEM((b_per_w, D), jnp.float32),
            pltpu.VMEM((b_per_w,), jnp.int32),
            pltpu.SemaphoreType.DMA,
        ],
    )
    return f(normed, p0, p1)


def _gather_rows(table, idx, nrows_total, sub=64):
    b_per_w = nrows_total // _NW
    nsub = b_per_w // sub
    mesh = plsc.VectorSubcoreMesh(core_axis_name="c", subcore_axis_name="s")

    def body(table_hbm, idx_hbm, out_hbm, idx_v, rows_v, sem):
        wid = jax.lax.axis_index("s") * _NC + jax.lax.axis_index("c")
        base = wid * b_per_w
        for k in range(nsub):
            pltpu.sync_copy(idx_hbm.at[pl.ds(base + k * sub, sub)], idx_v)
            pltpu.async_copy(table_hbm.at[idx_v], rows_v, sem).wait()
            pltpu.sync_copy(rows_v, out_hbm.at[pl.ds(base + k * sub, sub)])

    f = pl.kernel(
        body,
        out_type=jax.ShapeDtypeStruct((nrows_total, D), jnp.float32),
        mesh=mesh,
        scratch_types=[
            pltpu.VMEM((sub,), jnp.int32),
            pltpu.VMEM((sub, D), jnp.float32),
            pltpu.SemaphoreType.DMA,
        ],
    )
    return f(table, idx)


def _expert_body(te_ref, used_ref, gx_ref, w1_ref, b1_ref, w2_ref, b2_ref,
                 o_ref):
    j = pl.program_id(0)

    @pl.when(j < used_ref[0, 0])
    def _compute():
        gx = gx_ref[...]
        m1out = jax.lax.dot_general(
            gx, w1_ref[0], (((1,), (1,)), ((), ())),
            preferred_element_type=jnp.float32) + b1_ref[0, 0, :]
        gate_part = jnp.minimum(m1out[:, :FF], 7.0)
        linear_part = jnp.clip(m1out[:, FF:], -7.0, 7.0)
        swish_gate = gate_part * jax.nn.sigmoid(1.702 * gate_part)
        activated = swish_gate * (linear_part + 1.0)
        o_ref[...] = jax.lax.dot_general(
            activated, w2_ref[0], (((1,), (1,)), ((), ())),
            preferred_element_type=jnp.float32) + b2_ref[0, 0, :]


def _experts(te, used, gx, mlp1_weight, mlp1_bias, mlp2_weight, mlp2_bias):
    grid_spec = pltpu.PrefetchScalarGridSpec(
        num_scalar_prefetch=2,
        grid=(NTILES,),
        in_specs=[
            pl.BlockSpec((TTC, D), lambda j, te, u: (j, 0)),
            pl.BlockSpec((1, 2 * FF, D), lambda j, te, u: (te[j, 0], 0, 0)),
            pl.BlockSpec((1, 1, 2 * FF), lambda j, te, u: (te[j, 0], 0, 0)),
            pl.BlockSpec((1, D, FF), lambda j, te, u: (te[j, 0], 0, 0)),
            pl.BlockSpec((1, 1, D), lambda j, te, u: (te[j, 0], 0, 0)),
        ],
        out_specs=pl.BlockSpec((TTC, D), lambda j, te, u: (j, 0)),
    )
    return pl.pallas_call(
        _expert_body,
        grid_spec=grid_spec,
        out_shape=jax.ShapeDtypeStruct((NSLOT, D), jnp.float32),
    )(te, used, gx,
      mlp1_weight, mlp1_bias.reshape(E, 1, 2 * FF),
      mlp2_weight, mlp2_bias.reshape(E, 1, D))


def _combine_body(x_ref, mf_ref, ga_ref, gb_ref, o_ref):
    wa = mf_ref[:, 0:1]
    wb = mf_ref[:, 1:2]
    o_ref[...] = x_ref[...] + wa * ga_ref[...] + wb * gb_ref[...]


def _combine(x, mf, gpair):
    return pl.pallas_call(
        _combine_body,
        grid=(NT,),
        in_specs=[
            pl.BlockSpec((TT, D), lambda t: (t, 0)),
            pl.BlockSpec((TT, 2), lambda t: (t, 0)),
            pl.BlockSpec((TT, D), lambda t: (t, 0)),
            pl.BlockSpec((TT, D), lambda t: (NT + t, 0)),
        ],
        out_specs=pl.BlockSpec((TT, D), lambda t: (t, 0)),
        out_shape=jax.ShapeDtypeStruct((T, D), jnp.float32),
    )(x, mf, gpair, gpair)


@jax.jit
def kernel(x, scale, gate_kernel, gate_bias, mlp1_weight, mlp1_bias,
           mlp2_weight, mlp2_bias):
    normed, mf, pos, te, used = _router(x, scale, gate_kernel, gate_bias)
    p0 = pos[:, 0]
    p1 = pos[:, 1]
    gx = _scatter_rows(normed, p0, p1)
    oslots = _experts(te, used, gx,
                      mlp1_weight, mlp1_bias, mlp2_weight, mlp2_bias)
    gpair = _gather_rows(oslots, jnp.concatenate([p0, p1]), NA)
    return _combine(x, mf, gpair)
